# baseline (speedup 1.0000x reference)
import functools

import jax
import jax.numpy as jnp
from jax import lax
from jax.experimental import pallas as pl
from jax.experimental.pallas import tpu as pltpu
from jax.experimental.pallas import tpu_sc as plsc

_ROWS_PER_BLOCK = 8
_EPS = 1e-10


def _label_gather(scores_flat, labels, n_cols):
    (b,) = labels.shape
    info = plsc.get_sparse_core_info()
    nc, ns, nl = info.num_cores, info.num_subcores, info.num_lanes
    nw = nc * ns
    bw = b // nw

    mesh = plsc.VectorSubcoreMesh(core_axis_name="c", subcore_axis_name="s")

    @functools.partial(
        pl.kernel,
        mesh=mesh,
        out_type=jax.ShapeDtypeStruct((b,), jnp.float32),
        scratch_types=[
            pltpu.VMEM((bw,), jnp.int32),
            pltpu.VMEM((bw,), jnp.int32),
            pltpu.VMEM((bw,), jnp.float32),
            pltpu.SemaphoreType.DMA,
        ],
    )
    def gather_k(scores_hbm, labels_hbm, out_hbm, lab_v, idx_v, val_v, sem):
        wid = lax.axis_index("s") * nc + lax.axis_index("c")
        base = wid * bw
        pltpu.sync_copy(labels_hbm.at[pl.ds(base, bw)], lab_v)
        for j in range(bw // nl):
            lab16 = lab_v[pl.ds(j * nl, nl)]
            row16 = base + j * nl + lax.iota(jnp.int32, nl)
            idx_v[pl.ds(j * nl, nl)] = row16 * n_cols + lab16
        pltpu.async_copy(scores_hbm.at[idx_v], val_v, sem).wait()
        pltpu.sync_copy(val_v, out_hbm.at[pl.ds(base, bw)])

    return gather_k(scores_flat, labels)


def _listmle_body(kmin_ref, lab_ref, sl_ref, x_ref, out_ref):
    i = pl.program_id(0)
    x = x_ref[...]
    lab = lab_ref[...]
    sl = sl_ref[...]
    r, n = x.shape
    neg_inf = jnp.float32(-jnp.inf)
    cols = jax.lax.broadcasted_iota(jnp.int32, (r, n), 1)

    m1 = jnp.max(x, axis=1, keepdims=True)
    eq1 = x == m1
    cnt1 = jnp.sum(eq1.astype(jnp.int32), axis=1, keepdims=True)
    v2 = jnp.max(jnp.where(eq1, neg_inf, x), axis=1, keepdims=True)
    cnt2 = jnp.sum((x == v2).astype(jnp.int32), axis=1, keepdims=True)
    v3 = jnp.max(jnp.where(x >= v2, neg_inf, x), axis=1, keepdims=True)
    s1 = m1
    s2 = jnp.where(cnt1 >= 2, m1, v2)
    s3 = jnp.where(cnt1 >= 3, m1, jnp.where(cnt1 + cnt2 >= 3, v2, v3))

    gt = jnp.sum((x > sl).astype(jnp.int32), axis=1, keepdims=True)
    tie = jnp.sum(((x == sl) & (cols < lab)).astype(jnp.int32),
                  axis=1, keepdims=True)
    rank = gt + tie

    c1 = jnp.exp(s1)
    c2 = c1 + jnp.exp(s2)
    c3 = c2 + jnp.exp(s3)
    csel = jnp.where(rank == 0, c1, jnp.where(rank == 1, c2, c3))
    logd = jnp.log(csel + jnp.float32(_EPS))
    kmin = jnp.minimum(kmin_ref[0, 0], 3)
    contrib = jnp.where(rank < kmin, logd - sl, jnp.float32(0.0))

    @pl.when(i == 0)
    def _():
        out_ref[0, 0] = jnp.float32(0.0)

    out_ref[0, 0] += jnp.sum(contrib)


def kernel(scores, labels, k):
    b, n = scores.shape
    r = _ROWS_PER_BLOCK
    g = b // r
    labels_i = labels.astype(jnp.int32)
    sl = _label_gather(scores.reshape(-1), labels_i, n)

    labels2 = labels_i.reshape(b, 1)
    sl2 = sl.reshape(b, 1)
    kmin = jnp.asarray(k, jnp.int32).reshape(1, 1)

    loss_sum = pl.pallas_call(
        _listmle_body,
        grid=(g,),
        in_specs=[
            pl.BlockSpec((1, 1), lambda i: (0, 0), memory_space=pltpu.SMEM),
            pl.BlockSpec((r, 1), lambda i: (i, 0)),
            pl.BlockSpec((r, 1), lambda i: (i, 0)),
            pl.BlockSpec((r, n), lambda i: (i, 0)),
        ],
        out_specs=pl.BlockSpec((1, 1), lambda i: (0, 0),
                               memory_space=pltpu.SMEM),
        out_shape=jax.ShapeDtypeStruct((1, 1), jnp.float32),
        compiler_params=pltpu.CompilerParams(
            dimension_semantics=("arbitrary",)),
    )(kmin, labels2, sl2, scores)

    return loss_sum[0, 0] / jnp.float32(b)


# speedup vs baseline: 2.6388x; 2.6388x over previous
import jax
import jax.numpy as jnp
from jax.experimental import pallas as pl
from jax.experimental.pallas import tpu as pltpu

_R = 8
_W = 512
_EPS = 1e-10


def _listmle_body(kmin_ref, labs_ref, lab_ref, x_ref, out_ref):
    i = pl.program_id(0)
    r = _R
    n = x_ref.shape[1]
    n_main = n // _W
    rem = n - n_main * _W
    neg_inf = jnp.float32(-jnp.inf)
    lab = lab_ref[...]

    iota128 = jax.lax.broadcasted_iota(jnp.int32, (1, 128), 1)
    sl_rows = []
    for rr in range(r):
        off = labs_ref[i, rr]
        base = pl.multiple_of((off // 128) * 128, 128)
        vec = x_ref[pl.ds(rr, 1), pl.ds(base, 128)]
        sl_rows.append(jnp.max(jnp.where(iota128 == off % 128, vec,
                                         jnp.float32(-jnp.inf))))
    sl = jnp.stack(sl_rows).reshape(r, 1)

    m1 = jnp.full((r, _W), neg_inf)
    m2 = jnp.full((r, _W), neg_inf)
    m3 = jnp.full((r, _W), neg_inf)
    for j in range(n_main):
        v = x_ref[:, j * _W:(j + 1) * _W]
        t1 = jnp.maximum(m1, v)
        u1 = jnp.minimum(m1, v)
        t2 = jnp.maximum(m2, u1)
        u2 = jnp.minimum(m2, u1)
        m3 = jnp.maximum(m3, u2)
        m1, m2 = t1, t2
    tail = x_ref[:, n_main * _W:n]

    z = jnp.concatenate([m1, m2, m3, tail], axis=1)

    s1 = jnp.max(z, axis=1, keepdims=True)
    eq1 = z == s1
    cnt1 = jnp.sum(eq1.astype(jnp.int32), axis=1, keepdims=True)
    v2 = jnp.max(jnp.where(eq1, neg_inf, z), axis=1, keepdims=True)
    cnt2 = jnp.sum((z == v2).astype(jnp.int32), axis=1, keepdims=True)
    v3 = jnp.max(jnp.where(z >= v2, neg_inf, z), axis=1, keepdims=True)
    s2 = jnp.where(cnt1 >= 2, s1, v2)
    s3 = jnp.where(cnt1 >= 3, s1, jnp.where(cnt1 + cnt2 >= 3, v2, v3))

    zgt = jnp.sum((z > sl).astype(jnp.int32), axis=1, keepdims=True)
    zeq = jnp.sum((z == sl).astype(jnp.int32), axis=1, keepdims=True)

    rank_scr = zgt
    need_exact = (zgt <= 2) & ((zeq >= 2) | (zgt + zeq >= 3))

    def exact_rank():
        iota_w = jax.lax.broadcasted_iota(jnp.int32, (r, _W), 1)
        gt = jnp.zeros((r, 1), jnp.int32)
        tie = jnp.zeros((r, 1), jnp.int32)
        for j in range(n_main):
            v = x_ref[:, j * _W:(j + 1) * _W]
            cols = iota_w + (j * _W)
            gt = gt + jnp.sum((v > sl).astype(jnp.int32), axis=1,
                              keepdims=True)
            tie = tie + jnp.sum(((v == sl) & (cols < lab)).astype(jnp.int32),
                                axis=1, keepdims=True)
        vt = x_ref[:, n_main * _W:n]
        colst = jax.lax.broadcasted_iota(jnp.int32, (r, rem), 1) + n_main * _W
        gt = gt + jnp.sum((vt > sl).astype(jnp.int32), axis=1, keepdims=True)
        tie = tie + jnp.sum(((vt == sl) & (colst < lab)).astype(jnp.int32),
                            axis=1, keepdims=True)
        return gt + tie

    rank = jax.lax.cond(jnp.any(need_exact), exact_rank, lambda: rank_scr)

    c1 = jnp.exp(s1)
    c2 = c1 + jnp.exp(s2)
    c3 = c2 + jnp.exp(s3)
    csel = jnp.where(rank == 0, c1, jnp.where(rank == 1, c2, c3))
    logd = jnp.log(csel + jnp.float32(_EPS))
    kmin = jnp.minimum(kmin_ref[0, 0], 3)
    contrib = jnp.where(rank < kmin, logd - sl, jnp.float32(0.0))

    @pl.when(i == 0)
    def _():
        out_ref[0, 0] = jnp.float32(0.0)

    out_ref[0, 0] += jnp.sum(contrib)


def kernel(scores, labels, k):
    b, n = scores.shape
    g = b // _R
    labels_i = labels.astype(jnp.int32)
    labs2 = labels_i.reshape(g, _R)
    labels2 = labels_i.reshape(b, 1)
    kmin = jnp.asarray(k, jnp.int32).reshape(1, 1)

    loss_sum = pl.pallas_call(
        _listmle_body,
        grid=(g,),
        in_specs=[
            pl.BlockSpec((1, 1), lambda i: (0, 0), memory_space=pltpu.SMEM),
            pl.BlockSpec(memory_space=pltpu.SMEM),
            pl.BlockSpec((_R, 1), lambda i: (i, 0)),
            pl.BlockSpec((_R, n), lambda i: (i, 0)),
        ],
        out_specs=pl.BlockSpec((1, 1), lambda i: (0, 0),
                               memory_space=pltpu.SMEM),
        out_shape=jax.ShapeDtypeStruct((1, 1), jnp.float32),
        compiler_params=pltpu.CompilerParams(
            dimension_semantics=("arbitrary",)),
    )(kmin, labs2, labels2, scores)

    return loss_sum[0, 0] / jnp.float32(b)
